# baseline (speedup 1.0000x reference)
import functools
import jax
import jax.numpy as jnp
from jax import lax
from jax.experimental import pallas as pl
from jax.experimental.pallas import tpu as pltpu
from jax.experimental.pallas import tpu_sc as plsc

_B, _P, _D = 4096, 100, 128
_NC, _NS, _L = 2, 16, 16
_NW = _NC * _NS
_ROWS = _B // _NW
_NSLOT = 4
_VPR = _D // _L


def _sc_body(feat_hbm, pe_hbm, out_hbm, pe_v, b0, b1, b2, b3,
             si0, si1, si2, si3, so0, so1, so2, so3):
    bufs = (b0, b1, b2, b3)
    sin = (si0, si1, si2, si3)
    sout = (so0, so1, so2, so3)
    wid = lax.axis_index("s") * _NC + lax.axis_index("c")
    base = wid * _ROWS

    pltpu.sync_copy(pe_hbm, pe_v)

    def fetch(p, s):
        pltpu.async_copy(feat_hbm.at[p, pl.ds(base, _ROWS)], bufs[s], sin[s])

    def fetch_wait(s):
        pltpu.make_async_copy(feat_hbm.at[0, pl.ds(base, _ROWS)], bufs[s], sin[s]).wait()

    def flush(p, s):
        pltpu.async_copy(bufs[s], out_hbm.at[p, pl.ds(base, _ROWS)], sout[s])

    def flush_wait(s):
        pltpu.make_async_copy(bufs[s], out_hbm.at[0, pl.ds(base, _ROWS)], sout[s]).wait()

    fetch(0, 0)
    fetch(1, 1)

    def outer(c4, _):
        p0 = c4 * _NSLOT
        for b in range(_NSLOT):
            p = p0 + b
            sf = (b + 2) % _NSLOT

            @pl.when(p >= 2)
            def _(sf=sf):
                flush_wait(sf)

            @pl.when(p + 2 < _P)
            def _(p=p, sf=sf):
                fetch(p + 2, sf)

            pevs = [pe_v[p, pl.ds(l * _L, _L)] for l in range(_VPR)]
            fetch_wait(b)
            buf = bufs[b]

            @plsc.parallel_loop(0, _ROWS, 1, unroll=2)
            def _(r, _buf=buf, _pevs=pevs):
                for l in range(_VPR):
                    plsc.addupdate(_buf.at[r, pl.ds(l * _L, _L)], _pevs[l])

            flush(p, b)
        return ()

    lax.fori_loop(0, _P // _NSLOT, outer, ())
    flush_wait(2)
    flush_wait(3)


def kernel(feat_tokens, pos_emb):
    feat_t = jnp.transpose(feat_tokens, (1, 0, 2))
    mesh = plsc.VectorSubcoreMesh(core_axis_name="c", subcore_axis_name="s")
    run = functools.partial(
        pl.kernel,
        mesh=mesh,
        out_type=jax.ShapeDtypeStruct((_P, _B, _D), jnp.float32),
        scratch_types=(
            [pltpu.VMEM((_P, _D), jnp.float32)]
            + [pltpu.VMEM((_ROWS, _D), jnp.float32) for _ in range(_NSLOT)]
            + [pltpu.SemaphoreType.DMA for _ in range(2 * _NSLOT)]
        ),
    )(_sc_body)
    out_t = run(feat_t, pos_emb)
    return jnp.transpose(out_t, (1, 0, 2))


# speedup vs baseline: 1.0009x; 1.0009x over previous
import functools
import jax
import jax.numpy as jnp
from jax import lax
from jax.experimental import pallas as pl
from jax.experimental.pallas import tpu as pltpu
from jax.experimental.pallas import tpu_sc as plsc

_B, _P, _D = 4096, 100, 128
_NC, _NS, _L = 2, 16, 16
_NW = _NC * _NS
_ROWS = _B // _NW
_NSLOT = 5
_VPR = _D // _L


def _sc_body(feat_hbm, pe_hbm, out_hbm, pe_v, b0, b1, b2, b3, b4,
             si0, si1, si2, si3, si4, so0, so1, so2, so3, so4):
    bufs = (b0, b1, b2, b3, b4)
    sin = (si0, si1, si2, si3, si4)
    sout = (so0, so1, so2, so3, so4)
    wid = lax.axis_index("s") * _NC + lax.axis_index("c")
    base = wid * _ROWS

    pltpu.sync_copy(pe_hbm, pe_v)

    def fetch(p, s):
        pltpu.async_copy(feat_hbm.at[p, pl.ds(base, _ROWS)], bufs[s], sin[s])

    def fetch_wait(s):
        pltpu.make_async_copy(feat_hbm.at[0, pl.ds(base, _ROWS)], bufs[s], sin[s]).wait()

    def flush(p, s):
        pltpu.async_copy(bufs[s], out_hbm.at[p, pl.ds(base, _ROWS)], sout[s])

    def flush_wait(s):
        pltpu.make_async_copy(bufs[s], out_hbm.at[0, pl.ds(base, _ROWS)], sout[s]).wait()

    fetch(0, 0)
    fetch(1, 1)
    fetch(2, 2)

    def outer(c4, _):
        p0 = c4 * _NSLOT
        for b in range(_NSLOT):
            p = p0 + b
            sf = (b + 3) % _NSLOT

            @pl.when(p >= 2)
            def _(sf=sf):
                flush_wait(sf)

            @pl.when(p + 3 < _P)
            def _(p=p, sf=sf):
                fetch(p + 3, sf)

            pevs = [pe_v[p, pl.ds(l * _L, _L)] for l in range(_VPR)]
            fetch_wait(b)
            buf = bufs[b]

            @plsc.parallel_loop(0, _ROWS, 1, unroll=4)
            def _(r, _buf=buf, _pevs=pevs):
                for l in range(_VPR):
                    plsc.addupdate(_buf.at[r, pl.ds(l * _L, _L)], _pevs[l])

            flush(p, b)
        return ()

    lax.fori_loop(0, _P // _NSLOT, outer, ())
    flush_wait(3)
    flush_wait(4)


def kernel(feat_tokens, pos_emb):
    feat_t = jnp.transpose(feat_tokens, (1, 0, 2))
    mesh = plsc.VectorSubcoreMesh(core_axis_name="c", subcore_axis_name="s")
    run = functools.partial(
        pl.kernel,
        mesh=mesh,
        out_type=jax.ShapeDtypeStruct((_P, _B, _D), jnp.float32),
        scratch_types=(
            [pltpu.VMEM((_P, _D), jnp.float32)]
            + [pltpu.VMEM((_ROWS, _D), jnp.float32) for _ in range(_NSLOT)]
            + [pltpu.SemaphoreType.DMA for _ in range(2 * _NSLOT)]
        ),
    )(_sc_body)
    out_t = run(feat_t, pos_emb)
    return jnp.transpose(out_t, (1, 0, 2))
